# baseline (speedup 1.0000x reference)
import functools
import math

import jax
import jax.numpy as jnp
from jax import lax
from jax.experimental import pallas as pl
from jax.experimental.pallas import tpu as pltpu
from jax.experimental.pallas import tpu_sc as plsc

B = 1024
L = 200
E = 64
V = 1000000
N_IDS = B * L
NW = 32

SLAB = 256
NSLAB_FULL = 3906
TAIL_V0 = NSLAB_FULL * SLAB
TAIL_W = V - TAIL_V0
TAIL_WORKER = NSLAB_FULL % NW
KMAX = 123
HITCAP = 8192
IDXCHUNK = 4096
NIDXCHUNK = N_IDS // IDXCHUNK
OUTW = 128


def _gather_body(tab, idx, out, idxbuf, hv, hj, sv, sj, slabb, tailb, stag,
                 j2, sem_i, sem_s, sem_o):
    w = lax.axis_index("s") * 2 + lax.axis_index("c")
    lanes = lax.iota(jnp.int32, 16)

    def drain_scatter():
        pltpu.make_async_copy(out.at[pl.ds(0, 128)], stag.at[0], sem_o).wait()

    pltpu.async_copy(idx.at[pl.ds(0, IDXCHUNK)], idxbuf.at[0], sem_i)

    def chunk_body(c, ch):
        pltpu.make_async_copy(
            idx.at[pl.ds(0, IDXCHUNK)], idxbuf.at[c & 1], sem_i).wait()

        @pl.when(c + 1 < NIDXCHUNK)
        def _():
            pltpu.async_copy(idx.at[pl.ds((c + 1) * IDXCHUNK, IDXCHUNK)],
                             idxbuf.at[(c + 1) & 1], sem_i)

        def vec_body(g, ch2):
            vv = idxbuf[c & 1, pl.ds(g * 16, 16)]
            m = (((vv >> 8) & 31) == w) & (ch2 <= HITCAP - 16)
            jv = c * IDXCHUNK + g * 16 + lanes
            plsc.store_compressed(hv.at[pl.ds(ch2, 16)], vv, mask=m)
            plsc.store_compressed(hj.at[pl.ds(ch2, 16)], jv, mask=m)
            return ch2 + jnp.sum(m.astype(jnp.int32))

        return lax.fori_loop(0, IDXCHUNK // 16, vec_body, ch)

    c_hits = lax.fori_loop(0, NIDXCHUNK, chunk_body, 0)
    nhvec = (c_hits + 15) // 16

    def filter_hits(sid):
        def fvec(g, n):
            vv = hv[pl.ds(g * 16, 16)]
            jv = hj[pl.ds(g * 16, 16)]
            valid = ((g * 16 + lanes) < c_hits) & ((vv >> 8) == sid)
            plsc.store_compressed(sv.at[pl.ds(n, 16)], vv, mask=valid)
            plsc.store_compressed(sj.at[pl.ds(n, 16)], jv, mask=valid)
            return n + jnp.sum(valid.astype(jnp.int32))

        nh2 = lax.fori_loop(0, nhvec, fvec, 0)

        lastv = jnp.full((16,), jnp.maximum(nh2 - 1, 0), jnp.int32)
        dv = plsc.load_gather(sv, [lastv])
        dj = plsc.load_gather(sj, [lastv])
        for t in range(8):
            sv[pl.ds(nh2 + t * 16, 16)] = dv
            sj[pl.ds(nh2 + t * 16, 16)] = dj

        return nh2

    def extract_groups(nh2, gc, src_ref, width_mask):
        ngroups = (nh2 + 127) >> 7

        def ext_group(g2, gc2):
            b = gc2 & 1

            @pl.when(gc2 >= 2)
            def _():
                drain_scatter()

            jrow = j2.at[b]

            def tvec(t, _):
                off = g2 * 128 + t * 16
                vl = sv[pl.ds(off, 16)] & width_mask
                jrow[pl.ds(t * 16, 16)] = sj[pl.ds(off, 16)]
                rvec = t * 16 + lanes

                def erow(eb, __):
                    for u in range(8):
                        e = eb * 8 + u
                        ev = jnp.full((16,), e, jnp.int32)
                        val = plsc.load_gather(src_ref, [ev, vl])
                        plsc.store_scatter(stag.at[b], [rvec, ev], val)
                    return 0

                lax.fori_loop(0, 8, erow, 0)
                return 0

            lax.fori_loop(0, 8, tvec, 0)
            pltpu.async_copy(stag.at[b], out.at[j2.at[b]], sem_o)
            return gc2 + 1

        return lax.fori_loop(0, ngroups, ext_group, gc)

    pltpu.async_copy(tab.at[:, pl.ds(w * SLAB, SLAB)], slabb.at[0], sem_s)

    def slab_body(k, gc):
        sid = jnp.minimum(w + NW * k, NSLAB_FULL - 1)
        pltpu.make_async_copy(
            tab.at[:, pl.ds(0, SLAB)], slabb.at[k & 1], sem_s).wait()

        @pl.when(k + 1 < KMAX)
        def _():
            sidn = jnp.minimum(w + NW * (k + 1), NSLAB_FULL - 1)
            pltpu.async_copy(tab.at[:, pl.ds(sidn * SLAB, SLAB)],
                             slabb.at[(k + 1) & 1], sem_s)

        nh2 = filter_hits(sid)
        return extract_groups(nh2, gc, slabb.at[k & 1], SLAB - 1)

    gcnt = lax.fori_loop(0, KMAX, slab_body, 0)

    pltpu.sync_copy(tab.at[:, pl.ds(TAIL_V0, TAIL_W)], tailb)
    nh2t = filter_hits(NSLAB_FULL)
    gcnt = extract_groups(nh2t, gcnt, tailb, TAIL_W - 1)

    def drain_body(_, x):
        drain_scatter()
        return x

    lax.fori_loop(0, jnp.minimum(gcnt, 2), drain_body, 0)


def _sc_gather(tableT, idx):
    mesh = plsc.VectorSubcoreMesh(core_axis_name="c", subcore_axis_name="s")
    return pl.kernel(
        _gather_body,
        mesh=mesh,
        out_type=jax.ShapeDtypeStruct((N_IDS, OUTW), jnp.float32),
        scratch_types=[
            pltpu.VMEM((2, IDXCHUNK), jnp.int32),
            pltpu.VMEM((HITCAP + 16,), jnp.int32),
            pltpu.VMEM((HITCAP + 16,), jnp.int32),
            pltpu.VMEM((HITCAP + 144,), jnp.int32),
            pltpu.VMEM((HITCAP + 144,), jnp.int32),
            pltpu.VMEM((2, E, SLAB), jnp.float32),
            pltpu.VMEM((E, TAIL_W), jnp.float32),
            pltpu.VMEM((2, 128, OUTW), jnp.float32),
            pltpu.VMEM((2, 128), jnp.int32),
            pltpu.SemaphoreType.DMA,
            pltpu.SemaphoreType.DMA,
            pltpu.SemaphoreType.DMA,
        ],
        compiler_params=pltpu.CompilerParams(use_tc_tiling_on_sc=True,
                                            needs_layout_passes=False),
    )(tableT, idx)


BB = 8


def _post_body(x_ref, w_ref, b_ref, g_ref, beta_ref, o_ref):
    x = x_ref[...][:, :, :E]
    w = w_ref[...]
    zero = jnp.zeros((BB, 1, E), jnp.float32)
    x_prev = jnp.concatenate([zero, x[:, :-1, :]], axis=1)
    x_next = jnp.concatenate([x[:, 1:, :], zero], axis=1)
    y = x_prev * w[0] + x * w[1] + x_next * w[2] + b_ref[...][0]
    y = 0.5 * y * (1.0 + lax.erf(y * (1.0 / math.sqrt(2.0))))
    mean = jnp.mean(y, axis=-1, keepdims=True)
    d = y - mean
    var = jnp.mean(d * d, axis=-1, keepdims=True)
    normed = d * lax.rsqrt(var + 1e-5)
    normed = normed * g_ref[...][0] + beta_ref[...][0]
    l2 = jnp.sqrt(jnp.sum(normed * normed, axis=-1, keepdims=True))
    o_ref[...] = normed / jnp.maximum(l2, 1e-12)


def _tc_post(emb, conv_w, conv_b, ln_gamma, ln_beta):
    w = conv_w[:, 0, :].T
    return pl.pallas_call(
        _post_body,
        grid=(B // BB,),
        in_specs=[
            pl.BlockSpec((BB, L, 128), lambda i: (i, 0, 0)),
            pl.BlockSpec((3, E), lambda i: (0, 0)),
            pl.BlockSpec((1, E), lambda i: (0, 0)),
            pl.BlockSpec((1, E), lambda i: (0, 0)),
            pl.BlockSpec((1, E), lambda i: (0, 0)),
        ],
        out_specs=pl.BlockSpec((BB, L, E), lambda i: (i, 0, 0)),
        out_shape=jax.ShapeDtypeStruct((B, L, E), jnp.float32),
    )(emb, w, conv_b.reshape(1, E), ln_gamma.reshape(1, E),
      ln_beta.reshape(1, E))


def kernel(input_ids, table, conv_w, conv_b, ln_gamma, ln_beta):
    idx = input_ids.astype(jnp.int32).reshape(-1)
    emb = _sc_gather(table.T, idx).reshape(B, L, 128)
    return _tc_post(emb, conv_w, conv_b, ln_gamma, ln_beta)


# speedup vs baseline: 1.8558x; 1.8558x over previous
import functools
import math

import jax
import jax.numpy as jnp
from jax import lax
from jax.experimental import pallas as pl
from jax.experimental.pallas import tpu as pltpu
from jax.experimental.pallas import tpu_sc as plsc

B = 1024
L = 200
E = 64
V = 1000000
N_IDS = B * L
NW = 32

SLAB = 512
KMAX = 62
NCELL = KMAX * 16
HITCAP = 8192
BPAD = HITCAP + 96
TRASH = HITCAP + 80
IDXCHUNK = 2048
NIDXCHUNK = N_IDS // IDXCHUNK
OUTW = 128
TAIL_V0 = 999936
LAST_V0 = 999424


def _gather_body(tab, idx, out, idxbuf, counts, startb, bv, bj, slabb, tailb,
                 stag, j2, sem_i, sem_s, sem_o):
    w = lax.axis_index("s") * 2 + lax.axis_index("c")
    lanes = lax.iota(jnp.int32, 16)

    def drain_scatter():
        pltpu.make_async_copy(out.at[pl.ds(0, 64)], stag.at[0], sem_o).wait()

    def scal(x16):
        return jnp.max(x16)

    for i in range(NCELL // 16):
        counts[pl.ds(i * 16, 16)] = jnp.zeros((16,), jnp.int32)

    pltpu.async_copy(idx.at[pl.ds(0, IDXCHUNK)], idxbuf.at[0], sem_i)

    def chunkA(c, carry):
        pltpu.make_async_copy(
            idx.at[pl.ds(0, IDXCHUNK)], idxbuf.at[c & 1], sem_i).wait()

        @pl.when(c + 1 < NIDXCHUNK)
        def _():
            pltpu.async_copy(idx.at[pl.ds((c + 1) * IDXCHUNK, IDXCHUNK)],
                             idxbuf.at[(c + 1) & 1], sem_i)

        def vecA(g, acc):
            vv = idxbuf[c & 1, pl.ds(g * 16, 16)]
            m = ((vv >> 9) & 31) == w
            cell = (vv >> 14) * 16 + lanes
            inc = jnp.where(m, 1, 0).astype(jnp.int32)
            plsc.addupdate_scatter(counts, [cell], inc)
            return acc

        return lax.fori_loop(0, IDXCHUNK // 16, vecA, carry)

    lax.fori_loop(0, NIDXCHUNK, chunkA, 0)

    def pvec(i, carry):
        cvec = counts[pl.ds(i * 16, 16)]
        ics = plsc.cumsum(cvec)
        excl = ics - cvec + carry
        startb[pl.ds(i * 16, 16)] = excl
        counts[pl.ds(i * 16, 16)] = excl
        return carry + scal(ics)

    c_hits = lax.fori_loop(0, NCELL // 16, pvec, 0)
    startb[pl.ds(NCELL, 16)] = jnp.full((16,), c_hits, jnp.int32)

    pltpu.async_copy(idx.at[pl.ds(0, IDXCHUNK)], idxbuf.at[0], sem_i)

    def chunkB(c, carry):
        pltpu.make_async_copy(
            idx.at[pl.ds(0, IDXCHUNK)], idxbuf.at[c & 1], sem_i).wait()

        @pl.when(c + 1 < NIDXCHUNK)
        def _():
            pltpu.async_copy(idx.at[pl.ds((c + 1) * IDXCHUNK, IDXCHUNK)],
                             idxbuf.at[(c + 1) & 1], sem_i)

        def vecB(g, acc):
            vv = idxbuf[c & 1, pl.ds(g * 16, 16)]
            jv = c * IDXCHUNK + g * 16 + lanes
            m = ((vv >> 9) & 31) == w
            cell = (vv >> 14) * 16 + lanes
            pos = plsc.load_gather(counts, [cell])
            inc = jnp.where(m, 1, 0).astype(jnp.int32)
            plsc.addupdate_scatter(counts, [cell], inc)
            pos_eff = jnp.where(m, jnp.minimum(pos, HITCAP - 1),
                                TRASH + lanes)
            plsc.store_scatter(bv, [pos_eff], vv)
            plsc.store_scatter(bj, [pos_eff], jv)
            return acc

        return lax.fori_loop(0, IDXCHUNK // 16, vecB, carry)

    lax.fori_loop(0, NIDXCHUNK, chunkB, 0)

    lastp = jnp.full((16,), jnp.maximum(c_hits - 1, 0), jnp.int32)
    dvp = plsc.load_gather(bv, [lastp])
    djp = plsc.load_gather(bj, [lastp])
    cend = jnp.minimum(c_hits, HITCAP)
    for t in range(4):
        bv[pl.ds(cend + t * 16, 16)] = dvp
        bj[pl.ds(cend + t * 16, 16)] = djp

    def extract_range(lo, hi, gc, load_fn):
        n = jnp.maximum(hi - lo, 0)
        ngroups = (n + 63) >> 6

        def ext_group(g2, gc2):
            bsel = gc2 & 1

            @pl.when(gc2 >= 2)
            def _():
                drain_scatter()

            jrow = j2.at[bsel]
            for u in range(4):
                off = lo + g2 * 64 + u * 16
                vv = bv[pl.ds(off, 16)]
                jrow[pl.ds(u * 16, 16)] = bj[pl.ds(off, 16)]
                rvec = u * 16 + lanes

                def erow(eb, acc, vv=vv, rvec=rvec, bsel=bsel):
                    for uu in range(8):
                        e = eb * 8 + uu
                        ev = jnp.full((16,), e, jnp.int32)
                        val = load_fn(ev, vv)
                        plsc.store_scatter(stag.at[bsel], [rvec, ev], val)
                    return acc

                lax.fori_loop(0, 8, erow, 0)
            pltpu.async_copy(stag.at[bsel], out.at[j2.at[bsel]], sem_o)
            return gc2 + 1

        return lax.fori_loop(0, ngroups, ext_group, gc)

    def cell_bounds(k):
        lo = jnp.minimum(scal(plsc.load_gather(
            startb, [jnp.full((16,), k * 16, jnp.int32)])), HITCAP)
        hi = jnp.minimum(scal(plsc.load_gather(
            startb, [jnp.full((16,), k * 16 + 16, jnp.int32)])), HITCAP)
        return lo, hi

    pltpu.async_copy(tab.at[:, pl.ds(w * SLAB, SLAB)], slabb.at[0], sem_s)

    def slab_body(k, gc):
        pltpu.make_async_copy(
            tab.at[:, pl.ds(0, SLAB)], slabb.at[k & 1], sem_s).wait()

        @pl.when(k + 1 < KMAX - 1)
        def _():
            v0n = (w + NW * (k + 1)) * SLAB
            pltpu.async_copy(tab.at[:, pl.ds(v0n, SLAB)],
                             slabb.at[(k + 1) & 1], sem_s)

        lo, hi = cell_bounds(k)
        src = slabb.at[k & 1]

        def load_main(ev, vv, src=src):
            return plsc.load_gather(src, [ev, vv & (SLAB - 1)])

        return extract_range(lo, hi, gc, load_main)

    gcnt = lax.fori_loop(0, KMAX - 1, slab_body, 0)

    pltpu.sync_copy(tab.at[:, pl.ds(LAST_V0, SLAB)], slabb.at[0])
    pltpu.sync_copy(tab.at[:, pl.ds(TAIL_V0, 64)], tailb)
    lo61, hi61 = cell_bounds(KMAX - 1)

    def load_last(ev, vv):
        vl = vv - LAST_V0
        va = plsc.load_gather(slabb.at[0], [ev, jnp.minimum(vl, SLAB - 1)])
        vb = plsc.load_gather(tailb, [ev, jnp.clip(vl - SLAB, 0, 63)])
        return jnp.where(vl < SLAB, va, vb)

    gcnt = extract_range(lo61, hi61, gcnt, load_last)

    def drain_body(_, x):
        drain_scatter()
        return x

    lax.fori_loop(0, jnp.minimum(gcnt, 2), drain_body, 0)


def _sc_gather(tableT, idx):
    mesh = plsc.VectorSubcoreMesh(core_axis_name="c", subcore_axis_name="s")
    return pl.kernel(
        _gather_body,
        mesh=mesh,
        out_type=jax.ShapeDtypeStruct((N_IDS, OUTW), jnp.float32),
        scratch_types=[
            pltpu.VMEM((2, IDXCHUNK), jnp.int32),
            pltpu.VMEM((NCELL + 16,), jnp.int32),
            pltpu.VMEM((NCELL + 32,), jnp.int32),
            pltpu.VMEM((BPAD,), jnp.int32),
            pltpu.VMEM((BPAD,), jnp.int32),
            pltpu.VMEM((2, E, SLAB), jnp.float32),
            pltpu.VMEM((E, 64), jnp.float32),
            pltpu.VMEM((2, 64, OUTW), jnp.float32),
            pltpu.VMEM((2, 64), jnp.int32),
            pltpu.SemaphoreType.DMA,
            pltpu.SemaphoreType.DMA,
            pltpu.SemaphoreType.DMA,
        ],
        compiler_params=pltpu.CompilerParams(use_tc_tiling_on_sc=True,
                                             needs_layout_passes=False),
    )(tableT, idx)


BB = 8


def _post_body(x_ref, w_ref, b_ref, g_ref, beta_ref, o_ref):
    x = x_ref[...][:, :, :E]
    w = w_ref[...]
    zero = jnp.zeros((BB, 1, E), jnp.float32)
    x_prev = jnp.concatenate([zero, x[:, :-1, :]], axis=1)
    x_next = jnp.concatenate([x[:, 1:, :], zero], axis=1)
    y = x_prev * w[0] + x * w[1] + x_next * w[2] + b_ref[...][0]
    y = 0.5 * y * (1.0 + lax.erf(y * (1.0 / math.sqrt(2.0))))
    mean = jnp.mean(y, axis=-1, keepdims=True)
    d = y - mean
    var = jnp.mean(d * d, axis=-1, keepdims=True)
    normed = d * lax.rsqrt(var + 1e-5)
    normed = normed * g_ref[...][0] + beta_ref[...][0]
    l2 = jnp.sqrt(jnp.sum(normed * normed, axis=-1, keepdims=True))
    o_ref[...] = normed / jnp.maximum(l2, 1e-12)


def _tc_post(emb, conv_w, conv_b, ln_gamma, ln_beta):
    w = conv_w[:, 0, :].T
    return pl.pallas_call(
        _post_body,
        grid=(B // BB,),
        in_specs=[
            pl.BlockSpec((BB, L, 128), lambda i: (i, 0, 0)),
            pl.BlockSpec((3, E), lambda i: (0, 0)),
            pl.BlockSpec((1, E), lambda i: (0, 0)),
            pl.BlockSpec((1, E), lambda i: (0, 0)),
            pl.BlockSpec((1, E), lambda i: (0, 0)),
        ],
        out_specs=pl.BlockSpec((BB, L, E), lambda i: (i, 0, 0)),
        out_shape=jax.ShapeDtypeStruct((B, L, E), jnp.float32),
    )(emb, w, conv_b.reshape(1, E), ln_gamma.reshape(1, E),
      ln_beta.reshape(1, E))


def kernel(input_ids, table, conv_w, conv_b, ln_gamma, ln_beta):
    idx = input_ids.astype(jnp.int32).reshape(-1)
    emb = _sc_gather(table.T, idx).reshape(B, L, 128)
    return _tc_post(emb, conv_w, conv_b, ln_gamma, ln_beta)


# speedup vs baseline: 2.6827x; 1.4455x over previous
import functools
import math

import jax
import jax.numpy as jnp
from jax import lax
from jax.experimental import pallas as pl
from jax.experimental.pallas import tpu as pltpu
from jax.experimental.pallas import tpu_sc as plsc

B = 1024
L = 200
E = 64
V = 1000000
N_IDS = B * L

SLAB = 512
KMAX = 62
NCELL = KMAX * 16
HITCAP = 7424
BPAD = HITCAP + 96
TRASH = HITCAP + 80
RCAP = 64
RCELLS = 256
RSIZE = RCELLS * RCAP + 16
IDS_PER_SCAN = N_IDS // 16
IDXCHUNK = 640
NIDXCHUNK = IDS_PER_SCAN // IDXCHUNK
OUTW = 128
SGRP = 16
TAIL_V0 = 999936
LAST_V0 = 999424


def _gather_body(tab, idx, out, hbv, hbj, hbc, idxbuf, rbv, rbj, rcnt,
                 bv, bj, counts, startb, slabb, tailb, stag, j2,
                 sem_i, sem_s, sem_o):
    c = lax.axis_index("c")
    s = lax.axis_index("s")
    lanes = lax.iota(jnp.int32, 16)

    def drain_scatter():
        pltpu.make_async_copy(out.at[pl.ds(0, SGRP)], stag.at[0],
                              sem_o).wait()

    def scal(x16):
        return jnp.max(x16)

    for i in range(RCELLS // 16 + 1):
        rcnt[pl.ds(i * 16, 16)] = jnp.zeros((16,), jnp.int32)

    my_base = s * IDS_PER_SCAN
    pltpu.async_copy(idx.at[pl.ds(my_base, IDXCHUNK)], idxbuf.at[0], sem_i)

    def chunkR(ch, acc):
        pltpu.make_async_copy(
            idx.at[pl.ds(0, IDXCHUNK)], idxbuf.at[ch & 1], sem_i).wait()

        @pl.when(ch + 1 < NIDXCHUNK)
        def _():
            pltpu.async_copy(
                idx.at[pl.ds(my_base + (ch + 1) * IDXCHUNK, IDXCHUNK)],
                idxbuf.at[(ch + 1) & 1], sem_i)

        def vecR(g, acc2):
            vv = idxbuf[ch & 1, pl.ds(g * 16, 16)]
            jv = my_base + ch * IDXCHUNK + g * 16 + lanes
            slab = vv >> 9
            keep = (slab & 1) == c
            cell = ((slab >> 1) & 15) * 16 + lanes
            pos = plsc.load_gather(rcnt, [cell])
            inc = jnp.where(keep, 1, 0).astype(jnp.int32)
            plsc.addupdate_scatter(rcnt, [cell], inc)
            flat = cell * RCAP + jnp.minimum(pos, RCAP - 1)
            pos_eff = jnp.where(keep, flat, RCELLS * RCAP + lanes)
            plsc.store_scatter(rbv, [pos_eff], vv)
            plsc.store_scatter(rbj, [pos_eff], jv)
            return acc2

        return lax.fori_loop(0, IDXCHUNK // 16, vecR, acc)

    lax.fori_loop(0, NIDXCHUNK, chunkR, 0)

    w = c * 16 + s
    pltpu.sync_copy(rbv, hbv.at[w])
    pltpu.sync_copy(rbj, hbj.at[w])
    pltpu.sync_copy(rcnt, hbc.at[w])
    plsc.subcore_barrier()

    for p in range(16):
        pltpu.sync_copy(hbv.at[c * 16 + p, pl.ds(s * 1024, 1024)],
                        rbv.at[pl.ds(p * 1024, 1024)])
        pltpu.sync_copy(hbj.at[c * 16 + p, pl.ds(s * 1024, 1024)],
                        rbj.at[pl.ds(p * 1024, 1024)])
        pltpu.sync_copy(hbc.at[c * 16 + p, pl.ds(s * 16, 16)],
                        rcnt.at[pl.ds(p * 16, 16)])

    NVEC = 16 * 1024 // 16

    for i in range(NCELL // 16 + 1):
        counts[pl.ds(i * 16, 16)] = jnp.zeros((16,), jnp.int32)

    def veclocal(i, place):
        o = i * 16
        cellin = jnp.full((16,), o // RCAP, jnp.int32)
        cnt16 = jnp.minimum(plsc.load_gather(rcnt, [cellin]), RCAP)
        vv = rbv[pl.ds(o, 16)]
        valid = ((o % RCAP) + lanes) < cnt16
        k = jnp.clip(vv >> 14, 0, KMAX - 1)
        cellk = k * 16 + lanes
        inc = jnp.where(valid, 1, 0).astype(jnp.int32)
        if not place:
            plsc.addupdate_scatter(counts, [cellk], inc)
        else:
            jv = rbj[pl.ds(o, 16)]
            pos = plsc.load_gather(counts, [cellk])
            plsc.addupdate_scatter(counts, [cellk], inc)
            pos_eff = jnp.where(valid, jnp.minimum(pos, HITCAP - 1),
                                TRASH + lanes)
            plsc.store_scatter(bv, [pos_eff], vv)
            plsc.store_scatter(bj, [pos_eff], jv)

    lax.fori_loop(0, NVEC, lambda i, a: (veclocal(i, False), a)[1], 0)

    def pvec(i, carry):
        cvec = counts[pl.ds(i * 16, 16)]
        ics = plsc.cumsum(cvec)
        excl = ics - cvec + carry
        startb[pl.ds(i * 16, 16)] = excl
        counts[pl.ds(i * 16, 16)] = excl
        return carry + scal(ics)

    c_hits = lax.fori_loop(0, NCELL // 16, pvec, 0)
    startb[pl.ds(NCELL, 16)] = jnp.full((16,), c_hits, jnp.int32)

    lax.fori_loop(0, NVEC, lambda i, a: (veclocal(i, True), a)[1], 0)

    lastp = jnp.full((16,), jnp.maximum(c_hits - 1, 0), jnp.int32)
    dvp = plsc.load_gather(bv, [lastp])
    djp = plsc.load_gather(bj, [lastp])
    cend = jnp.minimum(c_hits, HITCAP)
    for t in range(4):
        bv[pl.ds(cend + t * 16, 16)] = dvp
        bj[pl.ds(cend + t * 16, 16)] = djp

    def extract_range(lo, hi, gc, load_fn):
        n = jnp.maximum(hi - lo, 0)
        ngroups = (n + SGRP - 1) >> 4

        def ext_group(g2, gc2):
            bsel = gc2 & 1

            @pl.when(gc2 >= 2)
            def _():
                drain_scatter()

            jrow = j2.at[bsel]
            for u in range(SGRP // 16):
                off = lo + g2 * SGRP + u * 16
                vv = bv[pl.ds(off, 16)]
                jrow[pl.ds(u * 16, 16)] = bj[pl.ds(off, 16)]
                rvec = u * 16 + lanes

                def erow(eb, acc, vv=vv, rvec=rvec, bsel=bsel):
                    for uu in range(8):
                        e = eb * 8 + uu
                        ev = jnp.full((16,), e, jnp.int32)
                        val = load_fn(ev, vv)
                        plsc.store_scatter(stag.at[bsel], [rvec, ev], val)
                    return acc

                lax.fori_loop(0, 8, erow, 0)
            pltpu.async_copy(stag.at[bsel], out.at[j2.at[bsel]], sem_o)
            return gc2 + 1

        return lax.fori_loop(0, ngroups, ext_group, gc)

    def cell_bounds(k):
        lo = jnp.minimum(scal(plsc.load_gather(
            startb, [jnp.full((16,), k * 16, jnp.int32)])), HITCAP)
        hi = jnp.minimum(scal(plsc.load_gather(
            startb, [jnp.full((16,), k * 16 + 16, jnp.int32)])), HITCAP)
        return lo, hi

    slab0 = c + 2 * s
    pltpu.async_copy(tab.at[:, pl.ds(slab0 * SLAB, SLAB)], slabb.at[0],
                     sem_s)

    def slab_body(k, gc):
        pltpu.make_async_copy(
            tab.at[:, pl.ds(0, SLAB)], slabb.at[k & 1], sem_s).wait()

        @pl.when(k + 1 < KMAX - 1)
        def _():
            v0n = (slab0 + 32 * (k + 1)) * SLAB
            pltpu.async_copy(tab.at[:, pl.ds(v0n, SLAB)],
                             slabb.at[(k + 1) & 1], sem_s)

        lo, hi = cell_bounds(k)
        src = slabb.at[k & 1]

        def load_main(ev, vv, src=src):
            return plsc.load_gather(src, [ev, vv & (SLAB - 1)])

        return extract_range(lo, hi, gc, load_main)

    gcnt = lax.fori_loop(0, KMAX - 1, slab_body, 0)

    pltpu.sync_copy(tab.at[:, pl.ds(LAST_V0, SLAB)], slabb.at[0])
    pltpu.sync_copy(tab.at[:, pl.ds(TAIL_V0, 64)], tailb)
    lo61, hi61 = cell_bounds(KMAX - 1)

    def load_last(ev, vv):
        vl = vv - LAST_V0
        va = plsc.load_gather(slabb.at[0], [ev, jnp.minimum(vl, SLAB - 1)])
        vb = plsc.load_gather(tailb, [ev, jnp.clip(vl - SLAB, 0, 63)])
        return jnp.where(vl < SLAB, va, vb)

    gcnt = extract_range(lo61, hi61, gcnt, load_last)

    def drain_body(_, x):
        drain_scatter()
        return x

    lax.fori_loop(0, jnp.minimum(gcnt, 2), drain_body, 0)


def _sc_gather(tableT, idx):
    mesh = plsc.VectorSubcoreMesh(core_axis_name="c", subcore_axis_name="s")
    return pl.kernel(
        _gather_body,
        mesh=mesh,
        out_type=[
            jax.ShapeDtypeStruct((N_IDS, OUTW), jnp.float32),
            jax.ShapeDtypeStruct((32, RSIZE), jnp.int32),
            jax.ShapeDtypeStruct((32, RSIZE), jnp.int32),
            jax.ShapeDtypeStruct((32, RCELLS + 16), jnp.int32),
        ],
        scratch_types=[
            pltpu.VMEM((2, IDXCHUNK), jnp.int32),
            pltpu.VMEM((RSIZE,), jnp.int32),
            pltpu.VMEM((RSIZE,), jnp.int32),
            pltpu.VMEM((RCELLS + 16,), jnp.int32),
            pltpu.VMEM((BPAD,), jnp.int32),
            pltpu.VMEM((BPAD,), jnp.int32),
            pltpu.VMEM((NCELL + 16,), jnp.int32),
            pltpu.VMEM((NCELL + 32,), jnp.int32),
            pltpu.VMEM((2, E, SLAB), jnp.float32),
            pltpu.VMEM((E, 64), jnp.float32),
            pltpu.VMEM((2, SGRP, OUTW), jnp.float32),
            pltpu.VMEM((2, SGRP), jnp.int32),
            pltpu.SemaphoreType.DMA,
            pltpu.SemaphoreType.DMA,
            pltpu.SemaphoreType.DMA,
        ],
        compiler_params=pltpu.CompilerParams(use_tc_tiling_on_sc=True,
                                             needs_layout_passes=False),
    )(tableT, idx)


BB = 8


def _post_body(x_ref, w_ref, b_ref, g_ref, beta_ref, o_ref):
    x = x_ref[...][:, :, :E]
    w = w_ref[...]
    zero = jnp.zeros((BB, 1, E), jnp.float32)
    x_prev = jnp.concatenate([zero, x[:, :-1, :]], axis=1)
    x_next = jnp.concatenate([x[:, 1:, :], zero], axis=1)
    y = x_prev * w[0] + x * w[1] + x_next * w[2] + b_ref[...][0]
    y = 0.5 * y * (1.0 + lax.erf(y * (1.0 / math.sqrt(2.0))))
    mean = jnp.mean(y, axis=-1, keepdims=True)
    d = y - mean
    var = jnp.mean(d * d, axis=-1, keepdims=True)
    normed = d * lax.rsqrt(var + 1e-5)
    normed = normed * g_ref[...][0] + beta_ref[...][0]
    l2 = jnp.sqrt(jnp.sum(normed * normed, axis=-1, keepdims=True))
    o_ref[...] = normed / jnp.maximum(l2, 1e-12)


def _tc_post(emb, conv_w, conv_b, ln_gamma, ln_beta):
    w = conv_w[:, 0, :].T
    return pl.pallas_call(
        _post_body,
        grid=(B // BB,),
        in_specs=[
            pl.BlockSpec((BB, L, 128), lambda i: (i, 0, 0)),
            pl.BlockSpec((3, E), lambda i: (0, 0)),
            pl.BlockSpec((1, E), lambda i: (0, 0)),
            pl.BlockSpec((1, E), lambda i: (0, 0)),
            pl.BlockSpec((1, E), lambda i: (0, 0)),
        ],
        out_specs=pl.BlockSpec((BB, L, E), lambda i: (i, 0, 0)),
        out_shape=jax.ShapeDtypeStruct((B, L, E), jnp.float32),
    )(emb, w, conv_b.reshape(1, E), ln_gamma.reshape(1, E),
      ln_beta.reshape(1, E))


def kernel(input_ids, table, conv_w, conv_b, ln_gamma, ln_beta):
    idx = input_ids.astype(jnp.int32).reshape(-1)
    emb = _sc_gather(table.T, idx)[0].reshape(B, L, 128)
    return _tc_post(emb, conv_w, conv_b, ln_gamma, ln_beta)


# speedup vs baseline: 2.8125x; 1.0484x over previous
import functools
import math

import jax
import jax.numpy as jnp
from jax import lax
from jax.experimental import pallas as pl
from jax.experimental.pallas import tpu as pltpu
from jax.experimental.pallas import tpu_sc as plsc

B = 1024
L = 200
E = 64
V = 1000000
N_IDS = B * L

SLAB = 512
KMAX = 62
NCELL = KMAX * 16
HITCAP = 7424
BPAD = HITCAP + 96
TRASH = HITCAP + 80
RCAP = 64
RCELLS = 256
RSIZE = RCELLS * RCAP + 16
IDS_PER_SCAN = N_IDS // 16
IDXCHUNK = 640
NIDXCHUNK = IDS_PER_SCAN // IDXCHUNK
OUTW = 128
SGRP = 16
TAIL_V0 = 999936
LAST_V0 = 999424


def _gather_body(tab, idx, out, hbv, hbj, hbc, idxbuf, rbv, rbj, rcnt,
                 bv, bj, counts, startb, slabb, tailb, stag, j2,
                 sem_i, sem_s, sem_o):
    c = lax.axis_index("c")
    s = lax.axis_index("s")
    lanes = lax.iota(jnp.int32, 16)

    def drain_scatter():
        pltpu.make_async_copy(out.at[pl.ds(0, SGRP)], stag.at[0],
                              sem_o).wait()

    def scal(x16):
        return jnp.max(x16)

    for i in range(RCELLS // 16 + 1):
        rcnt[pl.ds(i * 16, 16)] = jnp.zeros((16,), jnp.int32)

    my_base = s * IDS_PER_SCAN
    pltpu.async_copy(idx.at[pl.ds(my_base, IDXCHUNK)], idxbuf.at[0], sem_i)

    def chunkR(ch, acc):
        pltpu.make_async_copy(
            idx.at[pl.ds(0, IDXCHUNK)], idxbuf.at[ch & 1], sem_i).wait()

        @pl.when(ch + 1 < NIDXCHUNK)
        def _():
            pltpu.async_copy(
                idx.at[pl.ds(my_base + (ch + 1) * IDXCHUNK, IDXCHUNK)],
                idxbuf.at[(ch + 1) & 1], sem_i)

        def vecR(g, acc2):
            vv = idxbuf[ch & 1, pl.ds(g * 16, 16)]
            jv = my_base + ch * IDXCHUNK + g * 16 + lanes
            slab = vv >> 9
            keep = (slab & 1) == c
            cell = ((slab >> 1) & 15) * 16 + lanes
            pos = plsc.load_gather(rcnt, [cell])
            inc = jnp.where(keep, 1, 0).astype(jnp.int32)
            plsc.addupdate_scatter(rcnt, [cell], inc)
            flat = cell * RCAP + jnp.minimum(pos, RCAP - 1)
            pos_eff = jnp.where(keep, flat, RCELLS * RCAP + lanes)
            plsc.store_scatter(rbv, [pos_eff], vv)
            plsc.store_scatter(rbj, [pos_eff], jv)
            return acc2

        return lax.fori_loop(0, IDXCHUNK // 16, vecR, acc)

    lax.fori_loop(0, NIDXCHUNK, chunkR, 0)

    w = c * 16 + s
    pltpu.sync_copy(rbv, hbv.at[w])
    pltpu.sync_copy(rbj, hbj.at[w])
    pltpu.sync_copy(rcnt, hbc.at[w])
    plsc.subcore_barrier()

    for p in range(16):
        pltpu.sync_copy(hbv.at[c * 16 + p, pl.ds(s * 1024, 1024)],
                        rbv.at[pl.ds(p * 1024, 1024)])
        pltpu.sync_copy(hbj.at[c * 16 + p, pl.ds(s * 1024, 1024)],
                        rbj.at[pl.ds(p * 1024, 1024)])
        pltpu.sync_copy(hbc.at[c * 16 + p, pl.ds(s * 16, 16)],
                        rcnt.at[pl.ds(p * 16, 16)])

    NVEC = 16 * 1024 // 16

    for i in range(NCELL // 16 + 1):
        counts[pl.ds(i * 16, 16)] = jnp.zeros((16,), jnp.int32)

    def veclocal(i, place):
        o = i * 16
        cellin = jnp.full((16,), o // RCAP, jnp.int32)
        cnt16 = jnp.minimum(plsc.load_gather(rcnt, [cellin]), RCAP)
        vv = rbv[pl.ds(o, 16)]
        valid = ((o % RCAP) + lanes) < cnt16
        k = jnp.clip(vv >> 14, 0, KMAX - 1)
        cellk = k * 16 + lanes
        inc = jnp.where(valid, 1, 0).astype(jnp.int32)
        if not place:
            plsc.addupdate_scatter(counts, [cellk], inc)
        else:
            jv = rbj[pl.ds(o, 16)]
            pos = plsc.load_gather(counts, [cellk])
            plsc.addupdate_scatter(counts, [cellk], inc)
            pos_eff = jnp.where(valid, jnp.minimum(pos, HITCAP - 1),
                                TRASH + lanes)
            plsc.store_scatter(bv, [pos_eff], vv)
            plsc.store_scatter(bj, [pos_eff], jv)

    lax.fori_loop(0, NVEC, lambda i, a: (veclocal(i, False), a)[1], 0)

    def pvec(i, carry):
        cvec = counts[pl.ds(i * 16, 16)]
        ics = plsc.cumsum(cvec)
        excl = ics - cvec + carry
        startb[pl.ds(i * 16, 16)] = excl
        counts[pl.ds(i * 16, 16)] = excl
        return carry + scal(ics)

    c_hits = lax.fori_loop(0, NCELL // 16, pvec, 0)
    startb[pl.ds(NCELL, 16)] = jnp.full((16,), c_hits, jnp.int32)

    lax.fori_loop(0, NVEC, lambda i, a: (veclocal(i, True), a)[1], 0)

    lastp = jnp.full((16,), jnp.maximum(c_hits - 1, 0), jnp.int32)
    dvp = plsc.load_gather(bv, [lastp])
    djp = plsc.load_gather(bj, [lastp])
    cend = jnp.minimum(c_hits, HITCAP)
    for t in range(4):
        bv[pl.ds(cend + t * 16, 16)] = dvp
        bj[pl.ds(cend + t * 16, 16)] = djp

    def extract_range(lo, hi, gc, load_fn):
        n = jnp.maximum(hi - lo, 0)
        ngroups = (n + SGRP - 1) >> 4

        def ext_group(g2, gc2):
            bsel = gc2 & 1

            @pl.when(gc2 >= 2)
            def _():
                drain_scatter()

            jrow = j2.at[bsel]
            for u in range(SGRP // 16):
                off = lo + g2 * SGRP + u * 16
                vv = bv[pl.ds(off, 16)]
                jrow[pl.ds(u * 16, 16)] = bj[pl.ds(off, 16)]
                rvec = u * 16 + lanes

                def erow(eb, acc, vv=vv, rvec=rvec, bsel=bsel):
                    for uu in range(8):
                        e = eb * 8 + uu
                        ev = jnp.full((16,), e, jnp.int32)
                        val = load_fn(ev, vv)
                        plsc.store_scatter(stag.at[bsel], [rvec, ev], val)
                    return acc

                lax.fori_loop(0, 8, erow, 0)
            pltpu.async_copy(stag.at[bsel], out.at[j2.at[bsel]], sem_o)
            return gc2 + 1

        return lax.fori_loop(0, ngroups, ext_group, gc)

    def cell_bounds(k):
        lo = jnp.minimum(scal(plsc.load_gather(
            startb, [jnp.full((16,), k * 16, jnp.int32)])), HITCAP)
        hi = jnp.minimum(scal(plsc.load_gather(
            startb, [jnp.full((16,), k * 16 + 16, jnp.int32)])), HITCAP)
        return lo, hi

    slab0 = c + 2 * s
    pltpu.async_copy(tab.at[:, pl.ds(slab0 * SLAB, SLAB)], slabb.at[0],
                     sem_s)

    def slab_body(k, gc):
        pltpu.make_async_copy(
            tab.at[:, pl.ds(0, SLAB)], slabb.at[k & 1], sem_s).wait()

        @pl.when(k + 1 < KMAX - 1)
        def _():
            v0n = (slab0 + 32 * (k + 1)) * SLAB
            pltpu.async_copy(tab.at[:, pl.ds(v0n, SLAB)],
                             slabb.at[(k + 1) & 1], sem_s)

        lo, hi = cell_bounds(k)
        src = slabb.at[k & 1]

        def load_main(ev, vv, src=src):
            return plsc.load_gather(src, [ev, vv & (SLAB - 1)])

        return extract_range(lo, hi, gc, load_main)

    gcnt = lax.fori_loop(0, KMAX - 1, slab_body, 0)

    pltpu.sync_copy(tab.at[:, pl.ds(LAST_V0, SLAB)], slabb.at[0])
    pltpu.sync_copy(tab.at[:, pl.ds(TAIL_V0, 64)], tailb)
    lo61, hi61 = cell_bounds(KMAX - 1)

    def load_last(ev, vv):
        vl = vv - LAST_V0
        va = plsc.load_gather(slabb.at[0], [ev, jnp.minimum(vl, SLAB - 1)])
        vb = plsc.load_gather(tailb, [ev, jnp.clip(vl - SLAB, 0, 63)])
        return jnp.where(vl < SLAB, va, vb)

    gcnt = extract_range(lo61, hi61, gcnt, load_last)

    def drain_body(_, x):
        drain_scatter()
        return x

    lax.fori_loop(0, jnp.minimum(gcnt, 2), drain_body, 0)


def _sc_gather(tableT, idx):
    mesh = plsc.VectorSubcoreMesh(core_axis_name="c", subcore_axis_name="s")
    return pl.kernel(
        _gather_body,
        mesh=mesh,
        out_type=[
            jax.ShapeDtypeStruct((N_IDS, OUTW), jnp.float32),
            jax.ShapeDtypeStruct((32, RSIZE), jnp.int32),
            jax.ShapeDtypeStruct((32, RSIZE), jnp.int32),
            jax.ShapeDtypeStruct((32, RCELLS + 16), jnp.int32),
        ],
        scratch_types=[
            pltpu.VMEM((2, IDXCHUNK), jnp.int32),
            pltpu.VMEM((RSIZE,), jnp.int32),
            pltpu.VMEM((RSIZE,), jnp.int32),
            pltpu.VMEM((RCELLS + 16,), jnp.int32),
            pltpu.VMEM((BPAD,), jnp.int32),
            pltpu.VMEM((BPAD,), jnp.int32),
            pltpu.VMEM((NCELL + 16,), jnp.int32),
            pltpu.VMEM((NCELL + 32,), jnp.int32),
            pltpu.VMEM((2, E, SLAB), jnp.float32),
            pltpu.VMEM((E, 64), jnp.float32),
            pltpu.VMEM((2, SGRP, OUTW), jnp.float32),
            pltpu.VMEM((2, SGRP), jnp.int32),
            pltpu.SemaphoreType.DMA,
            pltpu.SemaphoreType.DMA,
            pltpu.SemaphoreType.DMA,
        ],
        compiler_params=pltpu.CompilerParams(use_tc_tiling_on_sc=True,
                                             needs_layout_passes=False),
    )(tableT, idx)


BB = 16


def _post_body(x_ref, w_ref, b_ref, g_ref, beta_ref, o_ref):
    x = x_ref[...][:, :, :E]
    w = w_ref[...]
    zero = jnp.zeros((BB, 1, E), jnp.float32)
    x_prev = jnp.concatenate([zero, x[:, :-1, :]], axis=1)
    x_next = jnp.concatenate([x[:, 1:, :], zero], axis=1)
    y = x_prev * w[0] + x * w[1] + x_next * w[2] + b_ref[...][0]
    y = 0.5 * y * (1.0 + lax.erf(y * (1.0 / math.sqrt(2.0))))
    mean = jnp.mean(y, axis=-1, keepdims=True)
    d = y - mean
    var = jnp.mean(d * d, axis=-1, keepdims=True)
    normed = d * lax.rsqrt(var + 1e-5)
    normed = normed * g_ref[...][0] + beta_ref[...][0]
    l2 = jnp.sqrt(jnp.sum(normed * normed, axis=-1, keepdims=True))
    o_ref[...] = normed / jnp.maximum(l2, 1e-12)


def _tc_post(emb, conv_w, conv_b, ln_gamma, ln_beta):
    w = conv_w[:, 0, :].T
    return pl.pallas_call(
        _post_body,
        grid=(B // BB,),
        in_specs=[
            pl.BlockSpec((BB, L, 128), lambda i: (i, 0, 0)),
            pl.BlockSpec((3, E), lambda i: (0, 0)),
            pl.BlockSpec((1, E), lambda i: (0, 0)),
            pl.BlockSpec((1, E), lambda i: (0, 0)),
            pl.BlockSpec((1, E), lambda i: (0, 0)),
        ],
        out_specs=pl.BlockSpec((BB, L, E), lambda i: (i, 0, 0)),
        out_shape=jax.ShapeDtypeStruct((B, L, E), jnp.float32),
    )(emb, w, conv_b.reshape(1, E), ln_gamma.reshape(1, E),
      ln_beta.reshape(1, E))


def kernel(input_ids, table, conv_w, conv_b, ln_gamma, ln_beta):
    idx = input_ids.astype(jnp.int32).reshape(-1)
    emb = _sc_gather(table.T, idx)[0].reshape(B, L, 128)
    return _tc_post(emb, conv_w, conv_b, ln_gamma, ln_beta)
